# Initial kernel scaffold; baseline (speedup 1.0000x reference)
import jax
import jax.numpy as jnp
from jax.experimental import pallas as pl


def kernel(x_world, center, half, svf_L0, svf_L1):
    raise NotImplementedError("write your pallas kernel here")


# speedup vs baseline: 43.3286x; 43.3286x over previous
import functools

import jax
import jax.numpy as jnp
from jax import lax
from jax.experimental import pallas as pl
from jax.experimental.pallas import tpu as pltpu
from jax.experimental.pallas import tpu_sc as plsc

NC = 2
NS = 16
NW = NC * NS
B = 512
NR = B // 128
NG = B // 16


def _build_quad(svf):
    g = svf[0]
    t = jnp.transpose(g, (1, 2, 3, 0))
    t = jnp.pad(t, ((0, 0), (0, 0), (0, 0), (0, 1)))
    tx = jnp.roll(t, -1, axis=2)
    ty = jnp.roll(t, -1, axis=1)
    txy = jnp.roll(tx, -1, axis=1)
    q = jnp.concatenate([t, tx, ty, txy], axis=-1)
    return q.reshape(-1, 16)


@functools.lru_cache(maxsize=4)
def _make_sc_kernel(G, dims0, dims1):
    PPT = G // NW
    NCH = PPT // B
    assert PPT * NW == G and NCH * B == PPT
    DIMS = (dims0, dims1)

    mesh = plsc.VectorSubcoreMesh(core_axis_name="c", subcore_axis_name="s")

    @functools.partial(
        pl.kernel,
        mesh=mesh,
        out_type=jax.ShapeDtypeStruct((G * 3,), jnp.float32),
        compiler_params=pltpu.CompilerParams(
            needs_layout_passes=False, use_tc_tiling_on_sc=False),
        scratch_types=[
            pltpu.VMEM((12, 16), jnp.float32),
            pltpu.VMEM((B,), jnp.float32),
            pltpu.VMEM((B,), jnp.float32),
            pltpu.VMEM((B,), jnp.float32),
            pltpu.VMEM((NR, 128), jnp.int32),
            pltpu.VMEM((NR, 128), jnp.int32),
            pltpu.VMEM((NR, 128), jnp.int32),
            pltpu.VMEM((NR, 128), jnp.int32),
            pltpu.VMEM((8 * B,), jnp.float32),
            pltpu.VMEM((8 * B,), jnp.float32),
            pltpu.VMEM((B, 16), jnp.float32),
            pltpu.VMEM((B, 16), jnp.float32),
            pltpu.VMEM((B, 16), jnp.float32),
            pltpu.VMEM((B, 16), jnp.float32),
            pltpu.VMEM((B * 16,), jnp.float32),
            pltpu.VMEM((3 * B,), jnp.float32),
            pltpu.SemaphoreType.DMA,
        ],
    )
    def sc_kernel(params_hbm, xs_hbm, ys_hbm, zs_hbm, t0_hbm, t1_hbm, out_hbm,
                  params_v, xv, yv, zv, i00, i01, i10, i11, pb0, pb1,
                  g00, g01, g10, g11, sbuf, res, sem):
        wid = lax.axis_index("s") * NC + lax.axis_index("c")
        tile_base = wid * PPT
        pltpu.sync_copy(params_hbm, params_v)
        iota = lax.broadcasted_iota(jnp.int32, (16,), 0)
        iota3 = iota * 3
        iota16 = iota * 16
        qpat = (iota // 4) * B
        tbls = (t0_hbm, t1_hbm)
        ibufs = ((i00, i01), (i10, i11))
        gbufs = ((g00, g01), (g10, g11))
        pbs = (pb0, pb1)

        def phase_a(lvl):
            D, H, W = DIMS[lvl]
            HW = H * W
            sx = params_v[6 * lvl + 0]
            ox = params_v[6 * lvl + 1]
            sy = params_v[6 * lvl + 2]
            oy = params_v[6 * lvl + 3]
            sz = params_v[6 * lvl + 4]
            oz = params_v[6 * lvl + 5]
            pb = pbs[lvl]

            for j in range(NR):
                def pa(g2, _c):
                    off = j * 128 + g2 * 16
                    xg = xv[pl.ds(off, 16)]
                    yg = yv[pl.ds(off, 16)]
                    zg = zv[pl.ds(off, 16)]
                    ix = jnp.minimum(jnp.maximum(xg * sx + ox, 0.0), W - 1.0)
                    iy = jnp.minimum(jnp.maximum(yg * sy + oy, 0.0), H - 1.0)
                    iz = jnp.minimum(jnp.maximum(zg * sz + oz, 0.0), D - 1.0)
                    x0 = jnp.minimum(ix.astype(jnp.int32), W - 2)
                    y0 = jnp.minimum(iy.astype(jnp.int32), H - 2)
                    z0 = jnp.minimum(iz.astype(jnp.int32), D - 2)
                    wx = ix - x0.astype(jnp.float32)
                    wy = iy - y0.astype(jnp.float32)
                    wz = iz - z0.astype(jnp.float32)
                    r0 = z0 * HW + y0 * W + x0
                    col = g2 * 16
                    ibufs[lvl][0][j, pl.ds(col, 16)] = r0
                    ibufs[lvl][1][j, pl.ds(col, 16)] = r0 + HW
                    mx = 1.0 - wx
                    my = 1.0 - wy
                    mz = 1.0 - wz
                    pA = my * mx
                    pB = my * wx
                    pC = wy * mx
                    pD = wy * wx
                    pb[pl.ds(0 * B + off, 16)] = pA * mz
                    pb[pl.ds(1 * B + off, 16)] = pB * mz
                    pb[pl.ds(2 * B + off, 16)] = pC * mz
                    pb[pl.ds(3 * B + off, 16)] = pD * mz
                    pb[pl.ds(4 * B + off, 16)] = pA * wz
                    pb[pl.ds(5 * B + off, 16)] = pB * wz
                    pb[pl.ds(6 * B + off, 16)] = pC * wz
                    pb[pl.ds(7 * B + off, 16)] = pD * wz
                    return _c
                lax.fori_loop(0, 128 // 16, pa, 0)

        def fire_dmas(lvl):
            handles = []
            for z in range(2):
                for j in range(NR):
                    handles.append(pltpu.async_copy(
                        tbls[lvl].at[ibufs[lvl][z].at[j]],
                        gbufs[lvl][z].at[pl.ds(j * 128, 128)],
                        sem))
            return handles

        def chunk(ci, carry):
            base = tile_base + ci * B
            pltpu.sync_copy(xs_hbm.at[pl.ds(base, B)], xv)
            pltpu.sync_copy(ys_hbm.at[pl.ds(base, B)], yv)
            pltpu.sync_copy(zs_hbm.at[pl.ds(base, B)], zv)
            phase_a(0)
            handles = fire_dmas(0)
            phase_a(1)
            handles += fire_dmas(1)
            for h in handles:
                h.wait()

            def pb_point(p, _c):
                w00 = plsc.load_gather(pb0, [qpat + p])
                w01 = plsc.load_gather(pb0, [qpat + (p + 4 * B)])
                w10 = plsc.load_gather(pb1, [qpat + p])
                w11 = plsc.load_gather(pb1, [qpat + (p + 4 * B)])
                r = w00 * g00[p] + w01 * g01[p] + w10 * g10[p] + w11 * g11[p]
                sbuf[pl.ds(p * 16, 16)] = r
                return _c
            lax.fori_loop(0, B, pb_point, 0)

            def pb_reduce(gi, _c):
                off = gi * 16
                b16 = off * 16 + iota16
                idxo = off * 3 + iota3
                for c in range(3):
                    s = (plsc.load_gather(sbuf, [b16 + c])
                         + plsc.load_gather(sbuf, [b16 + (c + 4)])
                         + plsc.load_gather(sbuf, [b16 + (c + 8)])
                         + plsc.load_gather(sbuf, [b16 + (c + 12)]))
                    plsc.store_scatter(res, [idxo + c], s)
                return _c
            lax.fori_loop(0, NG, pb_reduce, 0)

            pltpu.sync_copy(res, out_hbm.at[pl.ds(base * 3, 3 * B)])
            return carry

        lax.fori_loop(0, NCH, chunk, 0)

    return sc_kernel


def kernel(x_world, center, half, svf_L0, svf_L1):
    G = x_world.shape[0]
    xs = x_world[:, 0]
    ys = x_world[:, 1]
    zs = x_world[:, 2]
    t0 = _build_quad(svf_L0)
    t1 = _build_quad(svf_L1)
    denom = half + 1e-8
    rows = []
    for svf in (svf_L0, svf_L1):
        dW = svf.shape[4]
        dH = svf.shape[3]
        dD = svf.shape[2]
        for a, dim in ((0, dW), (1, dH), (2, dD)):
            s = 0.5 * (dim - 1) / denom[a]
            o = 0.5 * (dim - 1) * (1.0 - center[a] / denom[a])
            rows.append(s)
            rows.append(o)
    params = jnp.tile(jnp.stack(rows)[:, None], (1, 16)).astype(jnp.float32)
    dims0 = (svf_L0.shape[2], svf_L0.shape[3], svf_L0.shape[4])
    dims1 = (svf_L1.shape[2], svf_L1.shape[3], svf_L1.shape[4])
    fn = _make_sc_kernel(G, dims0, dims1)
    out = fn(params, xs, ys, zs, t0, t1)
    return out.reshape(G, 3)


# speedup vs baseline: 51.3289x; 1.1846x over previous
import functools

import jax
import jax.numpy as jnp
from jax import lax
from jax.experimental import pallas as pl
from jax.experimental.pallas import tpu as pltpu
from jax.experimental.pallas import tpu_sc as plsc

NC = 2
NS = 16
NW = NC * NS
B = 512
NR = B // 128
NG = B // 16


def _build_quad(svf):
    g = svf[0]
    t = jnp.transpose(g, (1, 2, 3, 0))
    t = jnp.pad(t, ((0, 0), (0, 0), (0, 0), (0, 1)))
    tx = jnp.roll(t, -1, axis=2)
    ty = jnp.roll(t, -1, axis=1)
    txy = jnp.roll(tx, -1, axis=1)
    q = jnp.concatenate([t, tx, ty, txy], axis=-1)
    return q.reshape(-1, 16)


@functools.lru_cache(maxsize=4)
def _make_sc_kernel(G, dims0, dims1):
    PPT = G // NW
    NCH = PPT // B
    assert PPT * NW == G and NCH * B == PPT and NCH % 2 == 0
    DIMS = (dims0, dims1)

    mesh = plsc.VectorSubcoreMesh(core_axis_name="c", subcore_axis_name="s")

    @functools.partial(
        pl.kernel,
        mesh=mesh,
        out_type=jax.ShapeDtypeStruct((G * 3,), jnp.float32),
        compiler_params=pltpu.CompilerParams(
            needs_layout_passes=False, use_tc_tiling_on_sc=False),
        scratch_types=[
            pltpu.VMEM((12, 16), jnp.float32),
            pltpu.VMEM((3, B), jnp.float32),
            pltpu.VMEM((3, B), jnp.float32),
            pltpu.VMEM((NR, 128), jnp.int32),
            pltpu.VMEM((NR, 128), jnp.int32),
            pltpu.VMEM((NR, 128), jnp.int32),
            pltpu.VMEM((NR, 128), jnp.int32),
            pltpu.VMEM((NR, 128), jnp.int32),
            pltpu.VMEM((NR, 128), jnp.int32),
            pltpu.VMEM((NR, 128), jnp.int32),
            pltpu.VMEM((NR, 128), jnp.int32),
            pltpu.VMEM((8 * B,), jnp.float32),
            pltpu.VMEM((8 * B,), jnp.float32),
            pltpu.VMEM((8 * B,), jnp.float32),
            pltpu.VMEM((8 * B,), jnp.float32),
            pltpu.VMEM((B, 16), jnp.float32),
            pltpu.VMEM((B, 16), jnp.float32),
            pltpu.VMEM((B, 16), jnp.float32),
            pltpu.VMEM((B, 16), jnp.float32),
            pltpu.VMEM((B, 16), jnp.float32),
            pltpu.VMEM((B, 16), jnp.float32),
            pltpu.VMEM((B, 16), jnp.float32),
            pltpu.VMEM((B, 16), jnp.float32),
            pltpu.VMEM((B * 16,), jnp.float32),
            pltpu.VMEM((3 * B,), jnp.float32),
            pltpu.SemaphoreType.DMA,
            pltpu.SemaphoreType.DMA,
            pltpu.SemaphoreType.DMA,
            pltpu.SemaphoreType.DMA,
        ],
    )
    def sc_kernel(params_hbm, xyz_hbm, t0_hbm, t1_hbm, out_hbm,
                  params_v, cv0, cv1,
                  i000, i001, i010, i011, i100, i101, i110, i111,
                  pb00, pb01, pb10, pb11,
                  g000, g001, g010, g011, g100, g101, g110, g111,
                  sbuf, res, semg0, semg1, semc0, semc1):
        wid = lax.axis_index("s") * NC + lax.axis_index("c")
        tile_base = wid * PPT
        pltpu.sync_copy(params_hbm, params_v)
        iota = lax.broadcasted_iota(jnp.int32, (16,), 0)
        iota3 = iota * 3
        iota16 = iota * 16
        qpat = (iota // 4) * B
        tbls = (t0_hbm, t1_hbm)
        coords = (cv0, cv1)
        ibufs = (((i000, i001), (i010, i011)), ((i100, i101), (i110, i111)))
        pbs = ((pb00, pb01), (pb10, pb11))
        gbufs = (((g000, g001), (g010, g011)), ((g100, g101), (g110, g111)))
        semg = (semg0, semg1)
        semc = (semc0, semc1)
        prm = [params_v[i] for i in range(12)]

        def fire_coords(ci, slot):
            base = tile_base + (ci % NCH) * B
            pltpu.async_copy(
                xyz_hbm.at[:, pl.ds(base, B)], coords[slot], semc[slot])

        def wait_coords(slot):
            pltpu.make_async_copy(
                xyz_hbm.at[:, pl.ds(0, B)], coords[slot], semc[slot]).wait()

        def phase_a(slot):
            cv = coords[slot]
            for lvl in range(2):
                D, H, W = DIMS[lvl]
                HW = H * W
                sx, ox, sy, oy, sz, oz = prm[6 * lvl:6 * lvl + 6]
                pb = pbs[slot][lvl]
                ib0 = ibufs[slot][lvl][0]
                ib1 = ibufs[slot][lvl][1]
                for j in range(NR):
                    @plsc.parallel_loop(0, 128 // 16, unroll=2)
                    def pa(g2):
                        off = j * 128 + g2 * 16
                        xg = cv[0, pl.ds(off, 16)]
                        yg = cv[1, pl.ds(off, 16)]
                        zg = cv[2, pl.ds(off, 16)]
                        ix = jnp.minimum(jnp.maximum(xg * sx + ox, 0.0), W - 1.0)
                        iy = jnp.minimum(jnp.maximum(yg * sy + oy, 0.0), H - 1.0)
                        iz = jnp.minimum(jnp.maximum(zg * sz + oz, 0.0), D - 1.0)
                        x0 = jnp.minimum(ix.astype(jnp.int32), W - 2)
                        y0 = jnp.minimum(iy.astype(jnp.int32), H - 2)
                        z0 = jnp.minimum(iz.astype(jnp.int32), D - 2)
                        wx = ix - x0.astype(jnp.float32)
                        wy = iy - y0.astype(jnp.float32)
                        wz = iz - z0.astype(jnp.float32)
                        r0 = z0 * HW + y0 * W + x0
                        col = g2 * 16
                        ib0[j, pl.ds(col, 16)] = r0
                        ib1[j, pl.ds(col, 16)] = r0 + HW
                        mx = 1.0 - wx
                        my = 1.0 - wy
                        mz = 1.0 - wz
                        pA = my * mx
                        pB = my * wx
                        pC = wy * mx
                        pD = wy * wx
                        pb[pl.ds(0 * B + off, 16)] = pA * mz
                        pb[pl.ds(1 * B + off, 16)] = pB * mz
                        pb[pl.ds(2 * B + off, 16)] = pC * mz
                        pb[pl.ds(3 * B + off, 16)] = pD * mz
                        pb[pl.ds(4 * B + off, 16)] = pA * wz
                        pb[pl.ds(5 * B + off, 16)] = pB * wz
                        pb[pl.ds(6 * B + off, 16)] = pC * wz
                        pb[pl.ds(7 * B + off, 16)] = pD * wz

        def fire_gathers(slot):
            for lvl in range(2):
                for z in range(2):
                    for j in range(NR):
                        pltpu.async_copy(
                            tbls[lvl].at[ibufs[slot][lvl][z].at[j]],
                            gbufs[slot][lvl][z].at[pl.ds(j * 128, 128)],
                            semg[slot])

        def wait_gathers(slot):
            for lvl in range(2):
                for z in range(2):
                    for j in range(NR):
                        pltpu.make_async_copy(
                            tbls[lvl].at[ibufs[slot][lvl][z].at[j]],
                            gbufs[slot][lvl][z].at[pl.ds(j * 128, 128)],
                            semg[slot]).wait()

        def phase_b(ci, slot):
            base = tile_base + ci * B
            pb0l, pb1l = pbs[slot]
            (g00, g01), (g10, g11) = gbufs[slot]

            @plsc.parallel_loop(0, B, unroll=4)
            def pb_point(p):
                w00 = plsc.load_gather(pb0l, [qpat + p])
                w01 = plsc.load_gather(pb0l, [qpat + (p + 4 * B)])
                w10 = plsc.load_gather(pb1l, [qpat + p])
                w11 = plsc.load_gather(pb1l, [qpat + (p + 4 * B)])
                r = w00 * g00[p] + w01 * g01[p] + w10 * g10[p] + w11 * g11[p]
                sbuf[pl.ds(p * 16, 16)] = r

            @plsc.parallel_loop(0, NG, unroll=2)
            def pb_reduce(gi):
                off = gi * 16
                b16 = off * 16 + iota16
                idxo = off * 3 + iota3
                for c in range(3):
                    s = (plsc.load_gather(sbuf, [b16 + c])
                         + plsc.load_gather(sbuf, [b16 + (c + 4)])
                         + plsc.load_gather(sbuf, [b16 + (c + 8)])
                         + plsc.load_gather(sbuf, [b16 + (c + 12)]))
                    plsc.store_scatter(res, [idxo + c], s)

            pltpu.sync_copy(res, out_hbm.at[pl.ds(base * 3, 3 * B)])

        fire_coords(0, 0)
        wait_coords(0)
        phase_a(0)
        fire_gathers(0)
        fire_coords(1, 1)

        def body(cc, carry):
            a = 2 * cc
            bch = a + 1
            wait_coords(1)
            phase_a(1)
            fire_gathers(1)
            fire_coords(a + 2, 0)
            wait_gathers(0)
            phase_b(a, 0)
            wait_coords(0)
            phase_a(0)
            fire_gathers(0)
            fire_coords(bch + 2, 1)
            wait_gathers(1)
            phase_b(bch, 1)
            return carry

        lax.fori_loop(0, NCH // 2, body, 0)
        wait_gathers(0)
        wait_coords(1)

    return sc_kernel


def kernel(x_world, center, half, svf_L0, svf_L1):
    G = x_world.shape[0]
    xyz = x_world.T
    t0 = _build_quad(svf_L0)
    t1 = _build_quad(svf_L1)
    denom = half + 1e-8
    rows = []
    for svf in (svf_L0, svf_L1):
        dW = svf.shape[4]
        dH = svf.shape[3]
        dD = svf.shape[2]
        for a, dim in ((0, dW), (1, dH), (2, dD)):
            s = 0.5 * (dim - 1) / denom[a]
            o = 0.5 * (dim - 1) * (1.0 - center[a] / denom[a])
            rows.append(s)
            rows.append(o)
    params = jnp.tile(jnp.stack(rows)[:, None], (1, 16)).astype(jnp.float32)
    dims0 = (svf_L0.shape[2], svf_L0.shape[3], svf_L0.shape[4])
    dims1 = (svf_L1.shape[2], svf_L1.shape[3], svf_L1.shape[4])
    fn = _make_sc_kernel(G, dims0, dims1)
    out = fn(params, xyz, t0, t1)
    return out.reshape(G, 3)


# speedup vs baseline: 52.7247x; 1.0272x over previous
import functools

import jax
import jax.numpy as jnp
from jax import lax
from jax.experimental import pallas as pl
from jax.experimental.pallas import tpu as pltpu
from jax.experimental.pallas import tpu_sc as plsc

NC = 2
NS = 16
NW = NC * NS
B = 512
NR = B // 128
NG = B // 16


def _build_quad(svf):
    g = svf[0]
    t = jnp.transpose(g, (1, 2, 3, 0))
    t = jnp.pad(t, ((0, 0), (0, 0), (0, 0), (0, 1)))
    tx = jnp.roll(t, -1, axis=2)
    ty = jnp.roll(t, -1, axis=1)
    txy = jnp.roll(tx, -1, axis=1)
    q = jnp.concatenate([t, tx, ty, txy], axis=-1)
    return q.reshape(-1, 16)


@functools.lru_cache(maxsize=4)
def _make_sc_kernel(G, dims0, dims1):
    PPT = G // NW
    NCH = PPT // B
    assert PPT * NW == G and NCH * B == PPT and NCH % 2 == 0
    DIMS = (dims0, dims1)

    mesh = plsc.VectorSubcoreMesh(core_axis_name="c", subcore_axis_name="s")

    @functools.partial(
        pl.kernel,
        mesh=mesh,
        out_type=jax.ShapeDtypeStruct((G * 3,), jnp.float32),
        compiler_params=pltpu.CompilerParams(
            needs_layout_passes=False, use_tc_tiling_on_sc=False),
        scratch_types=[
            pltpu.VMEM((12, 16), jnp.float32),
            pltpu.VMEM((3, B), jnp.float32),
            pltpu.VMEM((3, B), jnp.float32),
            pltpu.VMEM((NR, 128), jnp.int32),
            pltpu.VMEM((NR, 128), jnp.int32),
            pltpu.VMEM((NR, 128), jnp.int32),
            pltpu.VMEM((NR, 128), jnp.int32),
            pltpu.VMEM((NR, 128), jnp.int32),
            pltpu.VMEM((NR, 128), jnp.int32),
            pltpu.VMEM((NR, 128), jnp.int32),
            pltpu.VMEM((NR, 128), jnp.int32),
            pltpu.VMEM((8 * B,), jnp.float32),
            pltpu.VMEM((8 * B,), jnp.float32),
            pltpu.VMEM((8 * B,), jnp.float32),
            pltpu.VMEM((8 * B,), jnp.float32),
            pltpu.VMEM((B, 16), jnp.float32),
            pltpu.VMEM((B, 16), jnp.float32),
            pltpu.VMEM((B, 16), jnp.float32),
            pltpu.VMEM((B, 16), jnp.float32),
            pltpu.VMEM((B, 16), jnp.float32),
            pltpu.VMEM((B, 16), jnp.float32),
            pltpu.VMEM((B, 16), jnp.float32),
            pltpu.VMEM((B, 16), jnp.float32),
            pltpu.VMEM((B * 16,), jnp.float32),
            pltpu.VMEM((3 * B,), jnp.float32),
            pltpu.SemaphoreType.DMA,
            pltpu.SemaphoreType.DMA,
            pltpu.SemaphoreType.DMA,
            pltpu.SemaphoreType.DMA,
        ],
    )
    def sc_kernel(params_hbm, xyz_hbm, t0_hbm, t1_hbm, out_hbm,
                  params_v, cv0, cv1,
                  i000, i001, i010, i011, i100, i101, i110, i111,
                  pb00, pb01, pb10, pb11,
                  g000, g001, g010, g011, g100, g101, g110, g111,
                  sbuf, res, semg0, semg1, semc0, semc1):
        wid = lax.axis_index("s") * NC + lax.axis_index("c")
        tile_base = wid * PPT
        pltpu.sync_copy(params_hbm, params_v)
        iota = lax.broadcasted_iota(jnp.int32, (16,), 0)
        iota3 = iota * 3
        iota16 = iota * 16
        qpat = (iota // 4) * B
        tbls = (t0_hbm, t1_hbm)
        coords = (cv0, cv1)
        ibufs = (((i000, i001), (i010, i011)), ((i100, i101), (i110, i111)))
        pbs = ((pb00, pb01), (pb10, pb11))
        gbufs = (((g000, g001), (g010, g011)), ((g100, g101), (g110, g111)))
        semg = (semg0, semg1)
        semc = (semc0, semc1)
        prm = [params_v[i] for i in range(12)]

        def fire_coords(ci, slot):
            base = tile_base + (ci % NCH) * B
            pltpu.async_copy(
                xyz_hbm.at[:, pl.ds(base, B)], coords[slot], semc[slot])

        def wait_coords(slot):
            pltpu.make_async_copy(
                xyz_hbm.at[:, pl.ds(0, B)], coords[slot], semc[slot]).wait()

        def phase_a(slot):
            cv = coords[slot]
            for lvl in range(2):
                D, H, W = DIMS[lvl]
                HW = H * W
                sx, ox, sy, oy, sz, oz = prm[6 * lvl:6 * lvl + 6]
                pb = pbs[slot][lvl]
                ib0 = ibufs[slot][lvl][0]
                ib1 = ibufs[slot][lvl][1]
                for j in range(NR):
                    @plsc.parallel_loop(0, 128 // 16, unroll=2)
                    def pa(g2):
                        off = j * 128 + g2 * 16
                        xg = cv[0, pl.ds(off, 16)]
                        yg = cv[1, pl.ds(off, 16)]
                        zg = cv[2, pl.ds(off, 16)]
                        ix = jnp.minimum(jnp.maximum(xg * sx + ox, 0.0), W - 1.0)
                        iy = jnp.minimum(jnp.maximum(yg * sy + oy, 0.0), H - 1.0)
                        iz = jnp.minimum(jnp.maximum(zg * sz + oz, 0.0), D - 1.0)
                        x0 = jnp.minimum(ix.astype(jnp.int32), W - 2)
                        y0 = jnp.minimum(iy.astype(jnp.int32), H - 2)
                        z0 = jnp.minimum(iz.astype(jnp.int32), D - 2)
                        wx = ix - x0.astype(jnp.float32)
                        wy = iy - y0.astype(jnp.float32)
                        wz = iz - z0.astype(jnp.float32)
                        r0 = z0 * HW + y0 * W + x0
                        col = g2 * 16
                        ib0[j, pl.ds(col, 16)] = r0
                        ib1[j, pl.ds(col, 16)] = r0 + HW
                        mx = 1.0 - wx
                        my = 1.0 - wy
                        mz = 1.0 - wz
                        pA = my * mx
                        pB = my * wx
                        pC = wy * mx
                        pD = wy * wx
                        pb[pl.ds(0 * B + off, 16)] = pA * mz
                        pb[pl.ds(1 * B + off, 16)] = pB * mz
                        pb[pl.ds(2 * B + off, 16)] = pC * mz
                        pb[pl.ds(3 * B + off, 16)] = pD * mz
                        pb[pl.ds(4 * B + off, 16)] = pA * wz
                        pb[pl.ds(5 * B + off, 16)] = pB * wz
                        pb[pl.ds(6 * B + off, 16)] = pC * wz
                        pb[pl.ds(7 * B + off, 16)] = pD * wz

        def fire_gathers(slot):
            for lvl in range(2):
                for z in range(2):
                    for j in range(NR):
                        pltpu.async_copy(
                            tbls[lvl].at[ibufs[slot][lvl][z].at[j]],
                            gbufs[slot][lvl][z].at[pl.ds(j * 128, 128)],
                            semg[slot])

        def wait_gathers(slot):
            for lvl in range(2):
                for z in range(2):
                    for j in range(NR):
                        pltpu.make_async_copy(
                            tbls[lvl].at[ibufs[slot][lvl][z].at[j]],
                            gbufs[slot][lvl][z].at[pl.ds(j * 128, 128)],
                            semg[slot]).wait()

        def phase_b(ci, slot):
            base = tile_base + ci * B
            pb0l, pb1l = pbs[slot]
            (g00, g01), (g10, g11) = gbufs[slot]

            @plsc.parallel_loop(0, B, unroll=8)
            def pb_point(p):
                pvec = qpat + p
                pvec2 = pvec + 4 * B
                w00 = plsc.load_gather(pb0l, [pvec])
                w01 = plsc.load_gather(pb0l, [pvec2])
                w10 = plsc.load_gather(pb1l, [pvec])
                w11 = plsc.load_gather(pb1l, [pvec2])
                r = w00 * g00[p] + w01 * g01[p] + w10 * g10[p] + w11 * g11[p]
                sbuf[pl.ds(p * 16, 16)] = r

            @plsc.parallel_loop(0, NG, unroll=2)
            def pb_reduce(gi):
                off = gi * 16
                b16 = off * 16 + iota16
                idxo = off * 3 + iota3
                for c in range(3):
                    s = (plsc.load_gather(sbuf, [b16 + c])
                         + plsc.load_gather(sbuf, [b16 + (c + 4)])
                         + plsc.load_gather(sbuf, [b16 + (c + 8)])
                         + plsc.load_gather(sbuf, [b16 + (c + 12)]))
                    plsc.store_scatter(res, [idxo + c], s)

            pltpu.sync_copy(res, out_hbm.at[pl.ds(base * 3, 3 * B)])

        fire_coords(0, 0)
        wait_coords(0)
        phase_a(0)
        fire_gathers(0)
        fire_coords(1, 1)

        def body(cc, carry):
            a = 2 * cc
            bch = a + 1
            wait_coords(1)
            phase_a(1)
            fire_gathers(1)
            fire_coords(a + 2, 0)
            wait_gathers(0)
            phase_b(a, 0)
            wait_coords(0)
            phase_a(0)
            fire_gathers(0)
            fire_coords(bch + 2, 1)
            wait_gathers(1)
            phase_b(bch, 1)
            return carry

        lax.fori_loop(0, NCH // 2, body, 0)
        wait_gathers(0)
        wait_coords(1)

    return sc_kernel


def kernel(x_world, center, half, svf_L0, svf_L1):
    G = x_world.shape[0]
    xyz = x_world.T
    t0 = _build_quad(svf_L0)
    t1 = _build_quad(svf_L1)
    denom = half + 1e-8
    rows = []
    for svf in (svf_L0, svf_L1):
        dW = svf.shape[4]
        dH = svf.shape[3]
        dD = svf.shape[2]
        for a, dim in ((0, dW), (1, dH), (2, dD)):
            s = 0.5 * (dim - 1) / denom[a]
            o = 0.5 * (dim - 1) * (1.0 - center[a] / denom[a])
            rows.append(s)
            rows.append(o)
    params = jnp.tile(jnp.stack(rows)[:, None], (1, 16)).astype(jnp.float32)
    dims0 = (svf_L0.shape[2], svf_L0.shape[3], svf_L0.shape[4])
    dims1 = (svf_L1.shape[2], svf_L1.shape[3], svf_L1.shape[4])
    fn = _make_sc_kernel(G, dims0, dims1)
    out = fn(params, xyz, t0, t1)
    return out.reshape(G, 3)


# speedup vs baseline: 78.1710x; 1.4826x over previous
import functools

import jax
import jax.numpy as jnp
from jax import lax
from jax.experimental import pallas as pl
from jax.experimental.pallas import tpu as pltpu
from jax.experimental.pallas import tpu_sc as plsc

NC = 2
NS = 16
NW = NC * NS
B = 512
NR = B // 128
NG = B // 16


def _build_quad(svf):
    g = svf[0]
    t = jnp.transpose(g, (1, 2, 3, 0))
    t = jnp.pad(t, ((0, 0), (0, 0), (0, 0), (0, 1)))
    tx = jnp.roll(t, -1, axis=2)
    ty = jnp.roll(t, -1, axis=1)
    txy = jnp.roll(tx, -1, axis=1)
    q = jnp.concatenate([t, tx, ty, txy], axis=-1)
    return q.reshape(-1, 16)


@functools.lru_cache(maxsize=4)
def _make_sc_kernel(G, dims0, dims1):
    PPT = G // NW
    NCH = PPT // B
    assert PPT * NW == G and NCH * B == PPT and NCH % 2 == 0
    DIMS = (dims0, dims1)

    mesh = plsc.VectorSubcoreMesh(core_axis_name="c", subcore_axis_name="s")

    @functools.partial(
        pl.kernel,
        mesh=mesh,
        out_type=jax.ShapeDtypeStruct((G * 3,), jnp.float32),
        compiler_params=pltpu.CompilerParams(
            needs_layout_passes=False, use_tc_tiling_on_sc=False),
        scratch_types=[
            pltpu.VMEM((12, 16), jnp.float32),
            pltpu.VMEM((3, B), jnp.float32),
            pltpu.VMEM((3, B), jnp.float32),
            pltpu.VMEM((NR, 128), jnp.int32),
            pltpu.VMEM((NR, 128), jnp.int32),
            pltpu.VMEM((NR, 128), jnp.int32),
            pltpu.VMEM((NR, 128), jnp.int32),
            pltpu.VMEM((NR, 128), jnp.int32),
            pltpu.VMEM((NR, 128), jnp.int32),
            pltpu.VMEM((NR, 128), jnp.int32),
            pltpu.VMEM((NR, 128), jnp.int32),
            pltpu.VMEM((8 * B,), jnp.float32),
            pltpu.VMEM((8 * B,), jnp.float32),
            pltpu.VMEM((8 * B,), jnp.float32),
            pltpu.VMEM((8 * B,), jnp.float32),
            pltpu.VMEM((B, 16), jnp.float32),
            pltpu.VMEM((B, 16), jnp.float32),
            pltpu.VMEM((B, 16), jnp.float32),
            pltpu.VMEM((B, 16), jnp.float32),
            pltpu.VMEM((B, 16), jnp.float32),
            pltpu.VMEM((B, 16), jnp.float32),
            pltpu.VMEM((B, 16), jnp.float32),
            pltpu.VMEM((B, 16), jnp.float32),
            pltpu.VMEM((3 * B,), jnp.float32),
            pltpu.SemaphoreType.DMA,
            pltpu.SemaphoreType.DMA,
            pltpu.SemaphoreType.DMA,
            pltpu.SemaphoreType.DMA,
        ],
    )
    def sc_kernel(params_hbm, xyz_hbm, t0_hbm, t1_hbm, out_hbm,
                  params_v, cv0, cv1,
                  i000, i001, i010, i011, i100, i101, i110, i111,
                  pb00, pb01, pb10, pb11,
                  g000, g001, g010, g011, g100, g101, g110, g111,
                  res, semg0, semg1, semc0, semc1):
        wid = lax.axis_index("s") * NC + lax.axis_index("c")
        tile_base = wid * PPT
        pltpu.sync_copy(params_hbm, params_v)
        iota = lax.broadcasted_iota(jnp.int32, (16,), 0)
        iota3 = iota * 3
        cols = [jnp.full((16,), v, jnp.int32) for v in range(15)]
        tbls = (t0_hbm, t1_hbm)
        coords = (cv0, cv1)
        ibufs = (((i000, i001), (i010, i011)), ((i100, i101), (i110, i111)))
        pbs = ((pb00, pb01), (pb10, pb11))
        gbufs = (((g000, g001), (g010, g011)), ((g100, g101), (g110, g111)))
        semg = (semg0, semg1)
        semc = (semc0, semc1)
        prm = [params_v[i] for i in range(12)]

        def fire_coords(ci, slot):
            base = tile_base + (ci % NCH) * B
            pltpu.async_copy(
                xyz_hbm.at[:, pl.ds(base, B)], coords[slot], semc[slot])

        def wait_coords(slot):
            pltpu.make_async_copy(
                xyz_hbm.at[:, pl.ds(0, B)], coords[slot], semc[slot]).wait()

        def phase_a(slot):
            cv = coords[slot]
            for lvl in range(2):
                D, H, W = DIMS[lvl]
                HW = H * W
                sx, ox, sy, oy, sz, oz = prm[6 * lvl:6 * lvl + 6]
                pb = pbs[slot][lvl]
                ib0 = ibufs[slot][lvl][0]
                ib1 = ibufs[slot][lvl][1]
                for j in range(NR):
                    @plsc.parallel_loop(0, 128 // 16, unroll=2)
                    def pa(g2):
                        off = j * 128 + g2 * 16
                        xg = cv[0, pl.ds(off, 16)]
                        yg = cv[1, pl.ds(off, 16)]
                        zg = cv[2, pl.ds(off, 16)]
                        ix = jnp.minimum(jnp.maximum(xg * sx + ox, 0.0), W - 1.0)
                        iy = jnp.minimum(jnp.maximum(yg * sy + oy, 0.0), H - 1.0)
                        iz = jnp.minimum(jnp.maximum(zg * sz + oz, 0.0), D - 1.0)
                        x0 = jnp.minimum(ix.astype(jnp.int32), W - 2)
                        y0 = jnp.minimum(iy.astype(jnp.int32), H - 2)
                        z0 = jnp.minimum(iz.astype(jnp.int32), D - 2)
                        wx = ix - x0.astype(jnp.float32)
                        wy = iy - y0.astype(jnp.float32)
                        wz = iz - z0.astype(jnp.float32)
                        r0 = z0 * HW + y0 * W + x0
                        col = g2 * 16
                        ib0[j, pl.ds(col, 16)] = r0
                        ib1[j, pl.ds(col, 16)] = r0 + HW
                        mx = 1.0 - wx
                        my = 1.0 - wy
                        mz = 1.0 - wz
                        pA = my * mx
                        pB = my * wx
                        pC = wy * mx
                        pD = wy * wx
                        pb[pl.ds(0 * B + off, 16)] = pA * mz
                        pb[pl.ds(1 * B + off, 16)] = pB * mz
                        pb[pl.ds(2 * B + off, 16)] = pC * mz
                        pb[pl.ds(3 * B + off, 16)] = pD * mz
                        pb[pl.ds(4 * B + off, 16)] = pA * wz
                        pb[pl.ds(5 * B + off, 16)] = pB * wz
                        pb[pl.ds(6 * B + off, 16)] = pC * wz
                        pb[pl.ds(7 * B + off, 16)] = pD * wz

        def fire_gathers(slot):
            for lvl in range(2):
                for z in range(2):
                    for j in range(NR):
                        pltpu.async_copy(
                            tbls[lvl].at[ibufs[slot][lvl][z].at[j]],
                            gbufs[slot][lvl][z].at[pl.ds(j * 128, 128)],
                            semg[slot])

        def wait_gathers(slot):
            for lvl in range(2):
                for z in range(2):
                    for j in range(NR):
                        pltpu.make_async_copy(
                            tbls[lvl].at[ibufs[slot][lvl][z].at[j]],
                            gbufs[slot][lvl][z].at[pl.ds(j * 128, 128)],
                            semg[slot]).wait()

        def phase_b(ci, slot):
            base = tile_base + ci * B
            gb = gbufs[slot]
            pbsl = pbs[slot]

            @plsc.parallel_loop(0, NG, unroll=2)
            def pbg(gi):
                off = gi * 16
                rowv = off + iota
                acc0 = jnp.zeros((16,), jnp.float32)
                acc1 = jnp.zeros((16,), jnp.float32)
                acc2 = jnp.zeros((16,), jnp.float32)
                for lvl in range(2):
                    pbl = pbsl[lvl]
                    for z in range(2):
                        gz = gb[lvl][z]
                        for q in range(4):
                            w = pbl[pl.ds((z * 4 + q) * B + off, 16)]
                            v0 = plsc.load_gather(gz, [rowv, cols[4 * q + 0]])
                            v1 = plsc.load_gather(gz, [rowv, cols[4 * q + 1]])
                            v2 = plsc.load_gather(gz, [rowv, cols[4 * q + 2]])
                            acc0 = acc0 + v0 * w
                            acc1 = acc1 + v1 * w
                            acc2 = acc2 + v2 * w
                idxo = off * 3 + iota3
                plsc.store_scatter(res, [idxo], acc0)
                plsc.store_scatter(res, [idxo + 1], acc1)
                plsc.store_scatter(res, [idxo + 2], acc2)

            pltpu.sync_copy(res, out_hbm.at[pl.ds(base * 3, 3 * B)])

        fire_coords(0, 0)
        wait_coords(0)
        phase_a(0)
        fire_gathers(0)
        fire_coords(1, 1)

        def body(cc, carry):
            a = 2 * cc
            bch = a + 1
            wait_coords(1)
            phase_a(1)
            fire_gathers(1)
            fire_coords(a + 2, 0)
            wait_gathers(0)
            phase_b(a, 0)
            wait_coords(0)
            phase_a(0)
            fire_gathers(0)
            fire_coords(bch + 2, 1)
            wait_gathers(1)
            phase_b(bch, 1)
            return carry

        lax.fori_loop(0, NCH // 2, body, 0)
        wait_gathers(0)
        wait_coords(1)

    return sc_kernel


def kernel(x_world, center, half, svf_L0, svf_L1):
    G = x_world.shape[0]
    xyz = x_world.T
    t0 = _build_quad(svf_L0)
    t1 = _build_quad(svf_L1)
    denom = half + 1e-8
    rows = []
    for svf in (svf_L0, svf_L1):
        dW = svf.shape[4]
        dH = svf.shape[3]
        dD = svf.shape[2]
        for a, dim in ((0, dW), (1, dH), (2, dD)):
            s = 0.5 * (dim - 1) / denom[a]
            o = 0.5 * (dim - 1) * (1.0 - center[a] / denom[a])
            rows.append(s)
            rows.append(o)
    params = jnp.tile(jnp.stack(rows)[:, None], (1, 16)).astype(jnp.float32)
    dims0 = (svf_L0.shape[2], svf_L0.shape[3], svf_L0.shape[4])
    dims1 = (svf_L1.shape[2], svf_L1.shape[3], svf_L1.shape[4])
    fn = _make_sc_kernel(G, dims0, dims1)
    out = fn(params, xyz, t0, t1)
    return out.reshape(G, 3)
